# Initial kernel scaffold; baseline (speedup 1.0000x reference)
import jax
import jax.numpy as jnp
from jax.experimental import pallas as pl


def kernel(x_vals, msg_src, msg_dst, W, b):
    raise NotImplementedError("write your pallas kernel here")


# speedup vs baseline: 3.0229x; 3.0229x over previous
import functools

import jax
import jax.numpy as jnp
from jax import lax
from jax.experimental import pallas as pl
from jax.experimental.pallas import tpu as pltpu
from jax.experimental.pallas import tpu_sc as plsc

NC = 2
NS = 16
L = 16

NWIN = 16
CHUNK = 128
BLK = 1280
CAP = 1536


def _matmul(x, W, b, mb=640):
    m, d_in = x.shape
    d_out = W.shape[1]

    def body(x_ref, w_ref, b_ref, o_ref):
        o_ref[...] = (
            jnp.dot(x_ref[...], w_ref[...], preferred_element_type=jnp.float32)
            + b_ref[...]
        )

    return pl.pallas_call(
        body,
        grid=(m // mb,),
        in_specs=[
            pl.BlockSpec((mb, d_in), lambda i: (i, 0)),
            pl.BlockSpec((d_in, d_out), lambda i: (0, 0)),
            pl.BlockSpec((d_out,), lambda i: (0,)),
        ],
        out_specs=pl.BlockSpec((mb, d_out), lambda i: (i, 0)),
        out_shape=jax.ShapeDtypeStruct((m, d_out), jnp.float32),
    )(x, W, b)


def _make_msg_pass(m, e, d, variant=9):
    assert m % NWIN == 0
    win = m // NWIN
    cpr = 80
    assert win % cpr == 0 and cpr % 8 == 0
    nch = win // cpr
    kmax = (nch + NS - 1) // NS
    assert e % NS == 0
    eps = e // NS
    assert eps % BLK == 0
    nblk = eps // BLK

    mesh = plsc.VectorSubcoreMesh(core_axis_name="c", subcore_axis_name="s")

    @functools.partial(
        pl.kernel,
        out_type=jax.ShapeDtypeStruct((m, d), jnp.float32),
        mesh=mesh,
        scratch_types=[
            pltpu.VMEM((BLK,), jnp.int32),
            pltpu.VMEM((BLK,), jnp.int32),
            pltpu.VMEM((CAP,), jnp.int32),
            pltpu.VMEM((CAP,), jnp.int32),
            pltpu.VMEM((CHUNK,), jnp.int32),
            pltpu.VMEM((CHUNK,), jnp.int32),
            pltpu.VMEM((CHUNK, d), jnp.float32),
            pltpu.VMEM((cpr, d), jnp.float32),
            pltpu.VMEM_SHARED((win + 1, d), jnp.float32),
            pltpu.SemaphoreType.DMA,
        ],
        compiler_params=pltpu.CompilerParams(needs_layout_passes=False),
    )
    def msg_pass(tx_hbm, src_hbm, dst_hbm, out_hbm,
                 src_v, dst_v, gidx, lidx, gchunk, schunk, rows_v, zero_v,
                 acc, sem):
        c = lax.axis_index("c")
        s = lax.axis_index("s")
        zvec = jnp.zeros((L,), jnp.float32)

        def zrow(i, carry):
            for j in range(d // L):
                zero_v[i, pl.ds(j * L, L)] = zvec
            return carry

        if variant >= -2:
            lax.fori_loop(0, cpr, zrow, 0)
        if variant <= -1:
            return

        edge_base = s * eps

        def drain_chunk(i, carry):
            o = i * CHUNK
            for j in range(CHUNK // L):
                gchunk[pl.ds(j * L, L)] = gidx[pl.ds(o + j * L, L)]
                schunk[pl.ds(j * L, L)] = lidx[pl.ds(o + j * L, L)]
            pltpu.async_copy(tx_hbm.at[gchunk], rows_v, sem).wait()
            pltpu.sync_copy(rows_v, acc.at[schunk], add=True)
            return carry

        def window_body(wi, carry):
            w = wi * NC + c
            base = w * win

            for k in range(kmax):
                idx = s + k * NS

                @pl.when(idx < nch)
                def _zero_chunk():
                    pltpu.sync_copy(zero_v, acc.at[pl.ds(idx * cpr, cpr)])
            if variant >= 1:
                plsc.subcore_barrier()

            def block_body(bi, cnt):
                off = edge_base + bi * BLK
                pltpu.sync_copy(src_hbm.at[pl.ds(off, BLK)], src_v)
                pltpu.sync_copy(dst_hbm.at[pl.ds(off, BLK)], dst_v)

                def vec_body(j, cnt):
                    dv = dst_v[pl.ds(j * L, L)]
                    sv = src_v[pl.ds(j * L, L)]
                    if variant == 31:
                        basev = jnp.zeros((L,), jnp.int32)
                        hiv = jnp.full((L,), win, jnp.int32)
                    else:
                        basev = jnp.full((L,), base, jnp.int32)
                        hiv = jnp.full((L,), base + win, jnp.int32)
                    if variant == 32:
                        msk = dv < hiv
                    else:
                        msk = (dv >= basev) & (dv < hiv)
                    if variant == 33:
                        pos = jnp.full((L,), cnt, jnp.int32)
                    else:
                        csum = plsc.cumsum(msk.astype(jnp.int32))
                        pos = csum + jnp.full((L,), cnt - 1, jnp.int32)
                    if variant != 34:
                        plsc.store_scatter(gidx, [pos], sv, mask=msk)
                        plsc.store_scatter(lidx, [pos], dv - basev, mask=msk)
                    if variant == 35:
                        return cnt + 1
                    return cnt + jnp.sum(msk.astype(jnp.int32))

                if variant >= 3 or variant in (31, 32, 33, 34, 35):
                    cnt = lax.fori_loop(0, BLK // L, vec_body, cnt)
                nfull = cnt // CHUNK
                if 4 <= variant < 30:
                    lax.fori_loop(0, nfull, drain_chunk, 0)

                if 5 <= variant < 30:
                    @pl.when(nfull > 0)
                    def _move_leftover():
                        rem = nfull * CHUNK
                        for j in range(CHUNK // L):
                            gv = gidx[pl.ds(rem + j * L, L)]
                            lv = lidx[pl.ds(rem + j * L, L)]
                            gidx[pl.ds(j * L, L)] = gv
                            lidx[pl.ds(j * L, L)] = lv

                return cnt - nfull * CHUNK

            cnt = 0
            if variant >= 2:
                cnt = lax.fori_loop(0, nblk, block_body, 0)

            if 6 <= variant < 30:
                @pl.when(cnt > 0)
                def _final_drain():
                    for j in range(CHUNK // L):
                        gidx[pl.ds(cnt + j * L, L)] = jnp.zeros((L,),
                                                                jnp.int32)
                        lidx[pl.ds(cnt + j * L, L)] = jnp.full((L,), win,
                                                               jnp.int32)
                    nfin = (cnt + CHUNK - 1) // CHUNK
                    lax.fori_loop(0, nfin, drain_chunk, 0)

            if variant >= 1:
                plsc.subcore_barrier()
                for k in range(kmax):
                    idx = s + k * NS

                    @pl.when(idx < nch)
                    def _copy_chunk():
                        r = idx * cpr
                        pltpu.sync_copy(
                            acc.at[pl.ds(r, cpr)],
                            out_hbm.at[pl.ds(base + r, cpr)],
                        )
            return carry

        lax.fori_loop(0, NWIN // NC, window_body, 0)

    return msg_pass


def kernel(x_vals, msg_src, msg_dst, W, b):
    m, d = x_vals.shape[0], W.shape[1]
    tx = _matmul(x_vals, W, b)
    msg_pass = _make_msg_pass(m, msg_src.shape[0], d)
    return msg_pass(tx, msg_src, msg_dst)


# speedup vs baseline: 4.6563x; 1.5403x over previous
import functools

import jax
import jax.numpy as jnp
from jax import lax
from jax.experimental import pallas as pl
from jax.experimental.pallas import tpu as pltpu
from jax.experimental.pallas import tpu_sc as plsc

NC = 2
NS = 16
L = 16

NWIN = 16
CHUNK = 64
NB = 4
BURST = NB * CHUNK
BLK = 1280
CAP = 1792


def _matmul(x, W, b, mb=640):
    m, d_in = x.shape
    d_out = W.shape[1]

    def body(x_ref, w_ref, b_ref, o_ref):
        o_ref[...] = (
            jnp.dot(x_ref[...], w_ref[...], preferred_element_type=jnp.float32)
            + b_ref[...]
        )

    return pl.pallas_call(
        body,
        grid=(m // mb,),
        in_specs=[
            pl.BlockSpec((mb, d_in), lambda i: (i, 0)),
            pl.BlockSpec((d_in, d_out), lambda i: (0, 0)),
            pl.BlockSpec((d_out,), lambda i: (0,)),
        ],
        out_specs=pl.BlockSpec((mb, d_out), lambda i: (i, 0)),
        out_shape=jax.ShapeDtypeStruct((m, d_out), jnp.float32),
    )(x, W, b)


def _make_msg_pass(m, e, d):
    assert m % NWIN == 0
    win = m // NWIN
    cpr = 40
    assert win % cpr == 0 and cpr % 8 == 0
    nch = win // cpr
    kmax = (nch + NS - 1) // NS
    assert e % NS == 0
    eps = e // NS
    assert eps % BLK == 0
    nblk = eps // BLK

    mesh = plsc.VectorSubcoreMesh(core_axis_name="c", subcore_axis_name="s")

    @functools.partial(
        pl.kernel,
        out_type=jax.ShapeDtypeStruct((m, d), jnp.float32),
        mesh=mesh,
        scratch_types=[
            pltpu.VMEM((2, BLK), jnp.int32),
            pltpu.VMEM((2, BLK), jnp.int32),
            pltpu.VMEM((CAP,), jnp.int32),
            pltpu.VMEM((CAP,), jnp.int32),
            pltpu.VMEM((NB, CHUNK), jnp.int32),
            pltpu.VMEM((NB, CHUNK), jnp.int32),
            pltpu.VMEM((NB, CHUNK, d), jnp.float32),
            pltpu.VMEM((cpr, d), jnp.float32),
            pltpu.VMEM_SHARED((win + 1, d), jnp.float32),
            pltpu.SemaphoreType.DMA,
            pltpu.SemaphoreType.DMA,
        ],
        compiler_params=pltpu.CompilerParams(needs_layout_passes=False),
    )
    def msg_pass(tx_hbm, src_hbm, dst_hbm, out_hbm,
                 src_v, dst_v, gidx, lidx, gchunk, schunk, rows_v, zero_v,
                 acc, esem, gsem):
        c = lax.axis_index("c")
        s = lax.axis_index("s")
        zvec = jnp.zeros((L,), jnp.float32)

        def zrow(i, carry):
            for j in range(d // L):
                zero_v[i, pl.ds(j * L, L)] = zvec
            return carry

        lax.fori_loop(0, cpr, zrow, 0)

        edge_base = s * eps

        def start_block(bi, buf):
            off = edge_base + bi * BLK
            pltpu.async_copy(src_hbm.at[pl.ds(off, BLK)], src_v.at[buf],
                             esem)
            pltpu.async_copy(dst_hbm.at[pl.ds(off, BLK)], dst_v.at[buf],
                             esem)

        def wait_block(buf):
            pltpu.make_async_copy(src_hbm.at[pl.ds(0, BLK)], src_v.at[buf],
                                  esem).wait()
            pltpu.make_async_copy(dst_hbm.at[pl.ds(0, BLK)], dst_v.at[buf],
                                  esem).wait()

        def stage_chunk(q, o):
            for j in range(CHUNK // L):
                gchunk[q, pl.ds(j * L, L)] = gidx[pl.ds(o + j * L, L)]
                schunk[q, pl.ds(j * L, L)] = lidx[pl.ds(o + j * L, L)]

        def burst_body(t, carry):
            o = t * BURST
            for q in range(NB):
                stage_chunk(q, o + q * CHUNK)
                pltpu.async_copy(tx_hbm.at[gchunk.at[q]], rows_v.at[q],
                                 gsem)
            for q in range(NB):
                pltpu.make_async_copy(tx_hbm.at[gchunk.at[q]],
                                      rows_v.at[q], gsem).wait()
            for q in range(NB):
                pltpu.sync_copy(rows_v.at[q], acc.at[schunk.at[q]],
                                add=True)
            return carry

        def drain_one(i, carry):
            stage_chunk(0, i * CHUNK)
            pltpu.async_copy(tx_hbm.at[gchunk.at[0]], rows_v.at[0],
                             gsem).wait()
            pltpu.sync_copy(rows_v.at[0], acc.at[schunk.at[0]], add=True)
            return carry

        def window_body(wi, carry):
            w = wi * NC + c
            base = w * win

            for k in range(kmax):
                idx = s + k * NS

                @pl.when(idx < nch)
                def _zero_chunk():
                    pltpu.sync_copy(zero_v, acc.at[pl.ds(idx * cpr, cpr)])
            plsc.subcore_barrier()

            start_block(0, 0)

            def block_body(bi, cnt):
                par = lax.rem(bi, 2)
                wait_block(par)

                @pl.when(bi + 1 < nblk)
                def _prefetch():
                    start_block(bi + 1, 1 - par)

                def vec_body(j, cnt):
                    dv = dst_v[par, pl.ds(j * L, L)]
                    sv = src_v[par, pl.ds(j * L, L)]
                    basev = jnp.full((L,), base, jnp.int32)
                    hiv = jnp.full((L,), base + win, jnp.int32)
                    msk = (dv >= basev) & (dv < hiv)
                    csum = plsc.cumsum(msk.astype(jnp.int32))
                    pos = csum + jnp.full((L,), cnt - 1, jnp.int32)
                    plsc.store_scatter(gidx, [pos], sv, mask=msk)
                    plsc.store_scatter(lidx, [pos], dv - basev, mask=msk)
                    return cnt + jnp.sum(msk.astype(jnp.int32))

                cnt = lax.fori_loop(0, BLK // L, vec_body, cnt)

                nburst = cnt // BURST
                lax.fori_loop(0, nburst, burst_body, 0)

                @pl.when(nburst > 0)
                def _move_leftover():
                    rem = nburst * BURST
                    for j in range(BURST // L):
                        gv = gidx[pl.ds(rem + j * L, L)]
                        lv = lidx[pl.ds(rem + j * L, L)]
                        gidx[pl.ds(j * L, L)] = gv
                        lidx[pl.ds(j * L, L)] = lv

                return cnt - nburst * BURST

            cnt = lax.fori_loop(0, nblk, block_body, 0)

            @pl.when(cnt > 0)
            def _final_drain():
                for j in range(CHUNK // L):
                    gidx[pl.ds(cnt + j * L, L)] = jnp.zeros((L,), jnp.int32)
                    lidx[pl.ds(cnt + j * L, L)] = jnp.full((L,), win,
                                                           jnp.int32)
                nfin = (cnt + CHUNK - 1) // CHUNK
                lax.fori_loop(0, nfin, drain_one, 0)

            plsc.subcore_barrier()

            for k in range(kmax):
                idx = s + k * NS

                @pl.when(idx < nch)
                def _copy_chunk():
                    r = idx * cpr
                    pltpu.sync_copy(
                        acc.at[pl.ds(r, cpr)],
                        out_hbm.at[pl.ds(base + r, cpr)],
                    )
            return carry

        lax.fori_loop(0, NWIN // NC, window_body, 0)

    return msg_pass


def kernel(x_vals, msg_src, msg_dst, W, b):
    m, d = x_vals.shape[0], W.shape[1]
    tx = _matmul(x_vals, W, b)
    msg_pass = _make_msg_pass(m, msg_src.shape[0], d)
    return msg_pass(tx, msg_src, msg_dst)


# speedup vs baseline: 5.3195x; 1.1424x over previous
import functools

import jax
import jax.numpy as jnp
from jax import lax
from jax.experimental import pallas as pl
from jax.experimental.pallas import tpu as pltpu
from jax.experimental.pallas import tpu_sc as plsc

NC = 2
NS = 16
L = 16

NWIN = 16
CHUNK = 64
NB = 4
BURST = NB * CHUNK
BLK = 1280
CAP = 1792


def _matmul(x, W, b, mb=640):
    m, d_in = x.shape
    d_out = W.shape[1]

    def body(x_ref, w_ref, b_ref, o_ref):
        o_ref[...] = (
            jnp.dot(x_ref[...], w_ref[...], preferred_element_type=jnp.float32)
            + b_ref[...]
        )

    return pl.pallas_call(
        body,
        grid=(m // mb,),
        in_specs=[
            pl.BlockSpec((mb, d_in), lambda i: (i, 0)),
            pl.BlockSpec((d_in, d_out), lambda i: (0, 0)),
            pl.BlockSpec((d_out,), lambda i: (0,)),
        ],
        out_specs=pl.BlockSpec((mb, d_out), lambda i: (i, 0)),
        out_shape=jax.ShapeDtypeStruct((m, d_out), jnp.float32),
    )(x, W, b)


def _make_msg_pass(m, e, d):
    assert m % NWIN == 0
    win = m // NWIN
    cpr = 40
    assert win % cpr == 0 and cpr % 8 == 0
    nch = win // cpr
    kmax = (nch + NS - 1) // NS
    assert e % NS == 0
    eps = e // NS
    assert eps % BLK == 0
    nblk = eps // BLK

    mesh = plsc.VectorSubcoreMesh(core_axis_name="c", subcore_axis_name="s")

    @functools.partial(
        pl.kernel,
        out_type=jax.ShapeDtypeStruct((m, d), jnp.float32),
        mesh=mesh,
        scratch_types=[
            pltpu.VMEM((2, BLK), jnp.int32),
            pltpu.VMEM((2, BLK), jnp.int32),
            pltpu.VMEM((CAP,), jnp.int32),
            pltpu.VMEM((CAP,), jnp.int32),
            pltpu.VMEM((NB, CHUNK), jnp.int32),
            pltpu.VMEM((NB, CHUNK, d), jnp.float32),
            pltpu.VMEM((cpr, d), jnp.float32),
            pltpu.VMEM_SHARED((win + 1, d), jnp.float32),
            pltpu.SemaphoreType.DMA,
            pltpu.SemaphoreType.DMA,
            pltpu.SemaphoreType.DMA,
        ],
        compiler_params=pltpu.CompilerParams(needs_layout_passes=False),
    )
    def msg_pass(tx_hbm, src_hbm, dst_hbm, out_hbm,
                 src_v, dst_v, gidx, lidx, schunk, rows_v, zero_v,
                 acc, esem, gsem, ssem):
        c = lax.axis_index("c")
        s = lax.axis_index("s")
        zvec = jnp.zeros((L,), jnp.float32)

        def zrow(i, carry):
            for j in range(d // L):
                zero_v[i, pl.ds(j * L, L)] = zvec
            return carry

        lax.fori_loop(0, cpr, zrow, 0)

        edge_base = s * eps

        def start_block(bi, buf):
            off = edge_base + bi * BLK
            pltpu.async_copy(src_hbm.at[pl.ds(off, BLK)], src_v.at[buf],
                             esem)
            pltpu.async_copy(dst_hbm.at[pl.ds(off, BLK)], dst_v.at[buf],
                             esem)

        def wait_block(buf):
            pltpu.make_async_copy(src_hbm.at[pl.ds(0, BLK)], src_v.at[buf],
                                  esem).wait()
            pltpu.make_async_copy(dst_hbm.at[pl.ds(0, BLK)], dst_v.at[buf],
                                  esem).wait()

        def stage_sidx(q, o):
            for j in range(CHUNK // L):
                schunk[q, pl.ds(j * L, L)] = lidx[pl.ds(o + j * L, L)]

        def burst_body(t, carry):
            o = t * BURST
            for q in range(NB):
                stage_sidx(q, o + q * CHUNK)
                pltpu.async_copy(
                    tx_hbm.at[gidx.at[pl.ds(o + q * CHUNK, CHUNK)]],
                    rows_v.at[q], gsem)
            sdescs = []
            for q in range(NB):
                pltpu.make_async_copy(
                    tx_hbm.at[gidx.at[pl.ds(o + q * CHUNK, CHUNK)]],
                    rows_v.at[q], gsem).wait()
                sdescs.append(
                    pltpu.async_copy(rows_v.at[q], acc.at[schunk.at[q]],
                                     ssem, add=True))
            for sd in sdescs:
                sd.wait()
            return carry

        def drain_one(i, carry):
            o = i * CHUNK
            stage_sidx(0, o)
            pltpu.async_copy(tx_hbm.at[gidx.at[pl.ds(o, CHUNK)]],
                             rows_v.at[0], gsem).wait()
            pltpu.sync_copy(rows_v.at[0], acc.at[schunk.at[0]], add=True)
            return carry

        def window_body(wi, carry):
            w = wi * NC + c
            base = w * win

            for k in range(kmax):
                idx = s + k * NS

                @pl.when(idx < nch)
                def _zero_chunk():
                    pltpu.async_copy(zero_v, acc.at[pl.ds(idx * cpr, cpr)],
                                     esem)
            for k in range(kmax):
                idx = s + k * NS

                @pl.when(idx < nch)
                def _zero_wait():
                    pltpu.make_async_copy(
                        zero_v, acc.at[pl.ds(idx * cpr, cpr)], esem).wait()
            plsc.subcore_barrier()

            start_block(0, 0)

            def block_body(bi, cnt):
                par = lax.rem(bi, 2)
                wait_block(par)

                @pl.when(bi + 1 < nblk)
                def _prefetch():
                    start_block(bi + 1, 1 - par)

                def vec_body(j, cnt):
                    dv = dst_v[par, pl.ds(j * L, L)]
                    sv = src_v[par, pl.ds(j * L, L)]
                    basev = jnp.full((L,), base, jnp.int32)
                    hiv = jnp.full((L,), base + win, jnp.int32)
                    msk = (dv >= basev) & (dv < hiv)
                    csum = plsc.cumsum(msk.astype(jnp.int32))
                    pos = csum + jnp.full((L,), cnt - 1, jnp.int32)
                    plsc.store_scatter(gidx, [pos], sv, mask=msk)
                    plsc.store_scatter(lidx, [pos], dv - basev, mask=msk)
                    return cnt + jnp.sum(msk.astype(jnp.int32))

                cnt = lax.fori_loop(0, BLK // L, vec_body, cnt)

                nburst = cnt // BURST
                lax.fori_loop(0, nburst, burst_body, 0)

                @pl.when(nburst > 0)
                def _move_leftover():
                    rem = nburst * BURST
                    for j in range(BURST // L):
                        gv = gidx[pl.ds(rem + j * L, L)]
                        lv = lidx[pl.ds(rem + j * L, L)]
                        gidx[pl.ds(j * L, L)] = gv
                        lidx[pl.ds(j * L, L)] = lv

                return cnt - nburst * BURST

            cnt = lax.fori_loop(0, nblk, block_body, 0)

            @pl.when(cnt > 0)
            def _final_drain():
                for j in range(CHUNK // L):
                    gidx[pl.ds(cnt + j * L, L)] = jnp.zeros((L,), jnp.int32)
                    lidx[pl.ds(cnt + j * L, L)] = jnp.full((L,), win,
                                                           jnp.int32)
                nfin = (cnt + CHUNK - 1) // CHUNK
                lax.fori_loop(0, nfin, drain_one, 0)

            plsc.subcore_barrier()

            for k in range(kmax):
                idx = s + k * NS

                @pl.when(idx < nch)
                def _copy_chunk():
                    r = idx * cpr
                    pltpu.async_copy(
                        acc.at[pl.ds(r, cpr)],
                        out_hbm.at[pl.ds(base + r, cpr)], esem)
            for k in range(kmax):
                idx = s + k * NS

                @pl.when(idx < nch)
                def _copy_wait():
                    r = idx * cpr
                    pltpu.make_async_copy(
                        acc.at[pl.ds(r, cpr)],
                        out_hbm.at[pl.ds(base + r, cpr)], esem).wait()
            return carry

        lax.fori_loop(0, NWIN // NC, window_body, 0)

    return msg_pass


def kernel(x_vals, msg_src, msg_dst, W, b):
    m, d = x_vals.shape[0], W.shape[1]
    tx = _matmul(x_vals, W, b)
    msg_pass = _make_msg_pass(m, msg_src.shape[0], d)
    return msg_pass(tx, msg_src, msg_dst)


# speedup vs baseline: 5.6613x; 1.0643x over previous
import functools

import jax
import jax.numpy as jnp
from jax import lax
from jax.experimental import pallas as pl
from jax.experimental.pallas import tpu as pltpu
from jax.experimental.pallas import tpu_sc as plsc

NC = 2
NS = 16
L = 16

NWIN = 16
CHUNK = 64
NB = 4
BURST = NB * CHUNK
BLK = 1280
CAP = 1792


def _matmul(x, W, b, mb=640):
    m, d_in = x.shape
    d_out = W.shape[1]

    def body(x_ref, w_ref, b_ref, o_ref):
        o_ref[...] = (
            jnp.dot(x_ref[...], w_ref[...], preferred_element_type=jnp.float32)
            + b_ref[...]
        )

    return pl.pallas_call(
        body,
        grid=(m // mb,),
        in_specs=[
            pl.BlockSpec((mb, d_in), lambda i: (i, 0)),
            pl.BlockSpec((d_in, d_out), lambda i: (0, 0)),
            pl.BlockSpec((d_out,), lambda i: (0,)),
        ],
        out_specs=pl.BlockSpec((mb, d_out), lambda i: (i, 0)),
        out_shape=jax.ShapeDtypeStruct((m, d_out), jnp.float32),
    )(x, W, b)


def _make_msg_pass(m, e, d):
    assert m % NWIN == 0
    win = m // NWIN
    cpr = 40
    assert win % cpr == 0 and cpr % 8 == 0
    nch = win // cpr
    kmax = (nch + NS - 1) // NS
    assert e % NS == 0
    eps = e // NS
    assert eps % BLK == 0
    nblk = eps // BLK

    mesh = plsc.VectorSubcoreMesh(core_axis_name="c", subcore_axis_name="s")

    @functools.partial(
        pl.kernel,
        out_type=jax.ShapeDtypeStruct((m, d), jnp.float32),
        mesh=mesh,
        scratch_types=[
            pltpu.VMEM((2, BLK), jnp.int32),
            pltpu.VMEM((2, BLK), jnp.int32),
            pltpu.VMEM((CAP,), jnp.int32),
            pltpu.VMEM((CAP,), jnp.int32),
            pltpu.VMEM((NB, CHUNK), jnp.int32),
            pltpu.VMEM((NB, CHUNK, d), jnp.float32),
            pltpu.VMEM((cpr, d), jnp.float32),
            pltpu.VMEM_SHARED((win + 1, d), jnp.float32),
            pltpu.SemaphoreType.DMA,
            pltpu.SemaphoreType.DMA,
            pltpu.SemaphoreType.DMA,
        ],
        compiler_params=pltpu.CompilerParams(needs_layout_passes=False),
    )
    def msg_pass(tx_hbm, src_hbm, dst_hbm, out_hbm,
                 src_v, dst_v, gidx, lidx, schunk, rows_v, zero_v,
                 acc, esem, gsem, ssem):
        c = lax.axis_index("c")
        s = lax.axis_index("s")
        zvec = jnp.zeros((L,), jnp.float32)

        def zrow(i, carry):
            for j in range(d // L):
                zero_v[i, pl.ds(j * L, L)] = zvec
            return carry

        lax.fori_loop(0, cpr, zrow, 0)

        edge_base = s * eps

        def start_block(bi, buf):
            off = edge_base + bi * BLK
            pltpu.async_copy(src_hbm.at[pl.ds(off, BLK)], src_v.at[buf],
                             esem)
            pltpu.async_copy(dst_hbm.at[pl.ds(off, BLK)], dst_v.at[buf],
                             esem)

        def wait_block(buf):
            pltpu.make_async_copy(src_hbm.at[pl.ds(0, BLK)], src_v.at[buf],
                                  esem).wait()
            pltpu.make_async_copy(dst_hbm.at[pl.ds(0, BLK)], dst_v.at[buf],
                                  esem).wait()

        def stage_sidx(q, o):
            for j in range(CHUNK // L):
                schunk[q, pl.ds(j * L, L)] = lidx[pl.ds(o + j * L, L)]

        def burst_body(t, carry):
            o = t * BURST
            for q in range(NB):
                stage_sidx(q, o + q * CHUNK)
                pltpu.async_copy(
                    tx_hbm.at[gidx.at[pl.ds(o + q * CHUNK, CHUNK)]],
                    rows_v.at[q], gsem)
            sdescs = []
            for q in range(NB):
                pltpu.make_async_copy(
                    tx_hbm.at[gidx.at[pl.ds(o + q * CHUNK, CHUNK)]],
                    rows_v.at[q], gsem).wait()
                sdescs.append(
                    pltpu.async_copy(rows_v.at[q], acc.at[schunk.at[q]],
                                     ssem, add=True))
            for sd in sdescs:
                sd.wait()
            return carry

        def drain_one(i, carry):
            o = i * CHUNK
            stage_sidx(0, o)
            pltpu.async_copy(tx_hbm.at[gidx.at[pl.ds(o, CHUNK)]],
                             rows_v.at[0], gsem).wait()
            pltpu.sync_copy(rows_v.at[0], acc.at[schunk.at[0]], add=True)
            return carry

        def window_body(wi, carry):
            w = wi * NC + c
            base = w * win
            basev = jnp.full((L,), base, jnp.int32)
            winu = jnp.full((L,), win, jnp.uint32)

            for k in range(kmax):
                idx = s + k * NS

                @pl.when(idx < nch)
                def _zero_chunk():
                    pltpu.async_copy(zero_v, acc.at[pl.ds(idx * cpr, cpr)],
                                     esem)
            for k in range(kmax):
                idx = s + k * NS

                @pl.when(idx < nch)
                def _zero_wait():
                    pltpu.make_async_copy(
                        zero_v, acc.at[pl.ds(idx * cpr, cpr)], esem).wait()
            plsc.subcore_barrier()

            start_block(0, 0)

            def block_body(bi, cnt):
                par = lax.rem(bi, 2)
                wait_block(par)

                @pl.when(bi + 1 < nblk)
                def _prefetch():
                    start_block(bi + 1, 1 - par)

                def vec_body(j, cnt):
                    dv0 = dst_v[par, pl.ds(j * 2 * L, L)]
                    sv0 = src_v[par, pl.ds(j * 2 * L, L)]
                    dv1 = dst_v[par, pl.ds(j * 2 * L + L, L)]
                    sv1 = src_v[par, pl.ds(j * 2 * L + L, L)]
                    l0 = dv0 - basev
                    l1 = dv1 - basev
                    m0 = l0.astype(jnp.uint32) < winu
                    m1 = l1.astype(jnp.uint32) < winu
                    c0 = plsc.cumsum(m0.astype(jnp.int32))
                    c1 = plsc.cumsum(m1.astype(jnp.int32))
                    t0 = c0[L - 1]
                    pos0 = c0 + jnp.full((L,), cnt - 1, jnp.int32)
                    pos1 = c1 + jnp.full((L,), cnt + t0 - 1, jnp.int32)
                    plsc.store_scatter(gidx, [pos0], sv0, mask=m0)
                    plsc.store_scatter(lidx, [pos0], l0, mask=m0)
                    plsc.store_scatter(gidx, [pos1], sv1, mask=m1)
                    plsc.store_scatter(lidx, [pos1], l1, mask=m1)
                    return cnt + t0 + c1[L - 1]

                cnt = lax.fori_loop(0, BLK // (2 * L), vec_body, cnt)

                nburst = cnt // BURST
                lax.fori_loop(0, nburst, burst_body, 0)

                @pl.when(nburst > 0)
                def _move_leftover():
                    rem = nburst * BURST
                    for j in range(BURST // L):
                        gv = gidx[pl.ds(rem + j * L, L)]
                        lv = lidx[pl.ds(rem + j * L, L)]
                        gidx[pl.ds(j * L, L)] = gv
                        lidx[pl.ds(j * L, L)] = lv

                return cnt - nburst * BURST

            cnt = lax.fori_loop(0, nblk, block_body, 0)

            @pl.when(cnt > 0)
            def _final_drain():
                for j in range(CHUNK // L):
                    gidx[pl.ds(cnt + j * L, L)] = jnp.zeros((L,), jnp.int32)
                    lidx[pl.ds(cnt + j * L, L)] = jnp.full((L,), win,
                                                           jnp.int32)
                nfin = (cnt + CHUNK - 1) // CHUNK
                lax.fori_loop(0, nfin, drain_one, 0)

            plsc.subcore_barrier()

            for k in range(kmax):
                idx = s + k * NS

                @pl.when(idx < nch)
                def _copy_chunk():
                    r = idx * cpr
                    pltpu.async_copy(
                        acc.at[pl.ds(r, cpr)],
                        out_hbm.at[pl.ds(base + r, cpr)], esem)
            for k in range(kmax):
                idx = s + k * NS

                @pl.when(idx < nch)
                def _copy_wait():
                    r = idx * cpr
                    pltpu.make_async_copy(
                        acc.at[pl.ds(r, cpr)],
                        out_hbm.at[pl.ds(base + r, cpr)], esem).wait()
            return carry

        lax.fori_loop(0, NWIN // NC, window_body, 0)

    return msg_pass


def kernel(x_vals, msg_src, msg_dst, W, b):
    m, d = x_vals.shape[0], W.shape[1]
    tx = _matmul(x_vals, W, b)
    msg_pass = _make_msg_pass(m, msg_src.shape[0], d)
    return msg_pass(tx, msg_src, msg_dst)


# speedup vs baseline: 6.5398x; 1.1552x over previous
import functools

import jax
import jax.numpy as jnp
from jax import lax
from jax.experimental import pallas as pl
from jax.experimental.pallas import tpu as pltpu
from jax.experimental.pallas import tpu_sc as plsc

NC = 2
NS = 16
L = 16

NWIN = 16
CHUNK = 64
NB = 4
BURST = NB * CHUNK
BLK = 1280
CAP = 1792


def _matmul(x, W, b, mb=640):
    m, d_in = x.shape
    d_out = W.shape[1]

    def body(x_ref, w_ref, b_ref, o_ref):
        o_ref[...] = (
            jnp.dot(x_ref[...], w_ref[...], preferred_element_type=jnp.float32)
            + b_ref[...]
        )

    return pl.pallas_call(
        body,
        grid=(m // mb,),
        in_specs=[
            pl.BlockSpec((mb, d_in), lambda i: (i, 0)),
            pl.BlockSpec((d_in, d_out), lambda i: (0, 0)),
            pl.BlockSpec((d_out,), lambda i: (0,)),
        ],
        out_specs=pl.BlockSpec((mb, d_out), lambda i: (i, 0)),
        out_shape=jax.ShapeDtypeStruct((m, d_out), jnp.float32),
    )(x, W, b)


def _make_msg_pass(m, e, d):
    assert m % NWIN == 0
    win = m // NWIN
    cpr = 40
    assert win % cpr == 0 and cpr % 8 == 0
    nch = win // cpr
    kmax = (nch + NS - 1) // NS
    assert e % NS == 0
    eps = e // NS
    assert eps % BLK == 0
    nblk = eps // BLK

    mesh = plsc.VectorSubcoreMesh(core_axis_name="c", subcore_axis_name="s")

    @functools.partial(
        pl.kernel,
        out_type=jax.ShapeDtypeStruct((m, d), jnp.float32),
        mesh=mesh,
        scratch_types=[
            pltpu.VMEM((2, BLK), jnp.int32),
            pltpu.VMEM((2, BLK), jnp.int32),
            pltpu.VMEM((CAP,), jnp.int32),
            pltpu.VMEM((CAP,), jnp.int32),
            pltpu.VMEM((NB, CHUNK), jnp.int32),
            pltpu.VMEM((NB, CHUNK, d), jnp.float32),
            pltpu.VMEM((cpr, d), jnp.float32),
            pltpu.VMEM_SHARED((win + 1, d), jnp.float32),
            pltpu.SemaphoreType.DMA,
            pltpu.SemaphoreType.DMA,
            pltpu.SemaphoreType.DMA,
        ],
        compiler_params=pltpu.CompilerParams(needs_layout_passes=False),
    )
    def msg_pass(tx_hbm, src_hbm, dst_hbm, out_hbm,
                 src_v, dst_v, gidx, lidx, schunk, rows_v, zero_v,
                 acc, esem, gsem, ssem):
        c = lax.axis_index("c")
        s = lax.axis_index("s")
        zvec = jnp.zeros((L,), jnp.float32)

        def zrow(i, carry):
            for j in range(d // L):
                zero_v[i, pl.ds(j * L, L)] = zvec
            return carry

        lax.fori_loop(0, cpr, zrow, 0)

        edge_base = s * eps

        def start_block(bi, buf):
            off = edge_base + bi * BLK
            pltpu.async_copy(src_hbm.at[pl.ds(off, BLK)], src_v.at[buf],
                             esem)
            pltpu.async_copy(dst_hbm.at[pl.ds(off, BLK)], dst_v.at[buf],
                             esem)

        def wait_block(buf):
            pltpu.make_async_copy(src_hbm.at[pl.ds(0, BLK)], src_v.at[buf],
                                  esem).wait()
            pltpu.make_async_copy(dst_hbm.at[pl.ds(0, BLK)], dst_v.at[buf],
                                  esem).wait()

        def stage_sidx(q, o):
            for j in range(CHUNK // L):
                schunk[q, pl.ds(j * L, L)] = lidx[pl.ds(o + j * L, L)]

        def burst_body(t, carry):
            o = t * BURST
            for q in range(NB):
                stage_sidx(q, o + q * CHUNK)
                pltpu.async_copy(
                    tx_hbm.at[gidx.at[pl.ds(o + q * CHUNK, CHUNK)]],
                    rows_v.at[q], gsem)
            sdescs = []
            for q in range(NB):
                pltpu.make_async_copy(
                    tx_hbm.at[gidx.at[pl.ds(o + q * CHUNK, CHUNK)]],
                    rows_v.at[q], gsem).wait()
                sdescs.append(
                    pltpu.async_copy(rows_v.at[q], acc.at[schunk.at[q]],
                                     ssem, add=True))
            for sd in sdescs:
                sd.wait()
            return carry

        def drain_one(i, carry):
            o = i * CHUNK
            stage_sidx(0, o)
            pltpu.async_copy(tx_hbm.at[gidx.at[pl.ds(o, CHUNK)]],
                             rows_v.at[0], gsem).wait()
            pltpu.sync_copy(rows_v.at[0], acc.at[schunk.at[0]], add=True)
            return carry

        def window_body(wi, carry):
            w = wi * NC + c
            base = w * win
            basev = jnp.full((L,), base, jnp.int32)
            winu = jnp.full((L,), win, jnp.uint32)

            for k in range(kmax):
                idx = s + k * NS

                @pl.when(idx < nch)
                def _zero_chunk():
                    pltpu.async_copy(zero_v, acc.at[pl.ds(idx * cpr, cpr)],
                                     esem)
            for k in range(kmax):
                idx = s + k * NS

                @pl.when(idx < nch)
                def _zero_wait():
                    pltpu.make_async_copy(
                        zero_v, acc.at[pl.ds(idx * cpr, cpr)], esem).wait()
            plsc.subcore_barrier()

            start_block(0, 0)

            def block_body(bi, cnt):
                par = lax.rem(bi, 2)
                wait_block(par)

                @pl.when(bi + 1 < nblk)
                def _prefetch():
                    start_block(bi + 1, 1 - par)

                def vec_body(j, cnt):
                    U = 4
                    dv = [dst_v[par, pl.ds(j * U * L + u * L, L)]
                          for u in range(U)]
                    sv = [src_v[par, pl.ds(j * U * L + u * L, L)]
                          for u in range(U)]
                    lo = [dv[u] - basev for u in range(U)]
                    mk = [lo[u].astype(jnp.uint32) < winu for u in range(U)]
                    cs = [plsc.cumsum(mk[u].astype(jnp.int32))
                          for u in range(U)]
                    off = cnt - 1
                    for u in range(U):
                        pos = cs[u] + jnp.full((L,), off, jnp.int32)
                        plsc.store_scatter(gidx, [pos], sv[u], mask=mk[u])
                        plsc.store_scatter(lidx, [pos], lo[u], mask=mk[u])
                        off = off + cs[u][L - 1]
                    return off + 1

                cnt = lax.fori_loop(0, BLK // (4 * L), vec_body, cnt)

                nburst = cnt // BURST
                lax.fori_loop(0, nburst, burst_body, 0)

                @pl.when(nburst > 0)
                def _move_leftover():
                    rem = nburst * BURST
                    for j in range(BURST // L):
                        gv = gidx[pl.ds(rem + j * L, L)]
                        lv = lidx[pl.ds(rem + j * L, L)]
                        gidx[pl.ds(j * L, L)] = gv
                        lidx[pl.ds(j * L, L)] = lv

                return cnt - nburst * BURST

            cnt = lax.fori_loop(0, nblk, block_body, 0)

            @pl.when(cnt > 0)
            def _final_drain():
                for j in range(CHUNK // L):
                    gidx[pl.ds(cnt + j * L, L)] = jnp.zeros((L,), jnp.int32)
                    lidx[pl.ds(cnt + j * L, L)] = jnp.full((L,), win,
                                                           jnp.int32)
                nfin = (cnt + CHUNK - 1) // CHUNK
                lax.fori_loop(0, nfin, drain_one, 0)

            plsc.subcore_barrier()

            for k in range(kmax):
                idx = s + k * NS

                @pl.when(idx < nch)
                def _copy_chunk():
                    r = idx * cpr
                    pltpu.async_copy(
                        acc.at[pl.ds(r, cpr)],
                        out_hbm.at[pl.ds(base + r, cpr)], esem)
            for k in range(kmax):
                idx = s + k * NS

                @pl.when(idx < nch)
                def _copy_wait():
                    r = idx * cpr
                    pltpu.make_async_copy(
                        acc.at[pl.ds(r, cpr)],
                        out_hbm.at[pl.ds(base + r, cpr)], esem).wait()
            return carry

        lax.fori_loop(0, NWIN // NC, window_body, 0)

    return msg_pass


def kernel(x_vals, msg_src, msg_dst, W, b):
    m, d = x_vals.shape[0], W.shape[1]
    tx = _matmul(x_vals, W, b)
    msg_pass = _make_msg_pass(m, msg_src.shape[0], d)
    return msg_pass(tx, msg_src, msg_dst)


# speedup vs baseline: 6.7407x; 1.0307x over previous
import functools

import jax
import jax.numpy as jnp
from jax import lax
from jax.experimental import pallas as pl
from jax.experimental.pallas import tpu as pltpu
from jax.experimental.pallas import tpu_sc as plsc

NC = 2
NS = 16
L = 16

NWIN = 16
CHUNK = 64
NB = 4
BURST = NB * CHUNK
BLK = 1280
CAP = 1792


def _matmul(x, W, b, mb=640):
    m, d_in = x.shape
    d_out = W.shape[1]

    def body(x_ref, w_ref, b_ref, o_ref):
        o_ref[...] = (
            jnp.dot(x_ref[...], w_ref[...], preferred_element_type=jnp.float32)
            + b_ref[...]
        )

    return pl.pallas_call(
        body,
        grid=(m // mb,),
        in_specs=[
            pl.BlockSpec((mb, d_in), lambda i: (i, 0)),
            pl.BlockSpec((d_in, d_out), lambda i: (0, 0)),
            pl.BlockSpec((d_out,), lambda i: (0,)),
        ],
        out_specs=pl.BlockSpec((mb, d_out), lambda i: (i, 0)),
        out_shape=jax.ShapeDtypeStruct((m, d_out), jnp.float32),
    )(x, W, b)


def _make_msg_pass(m, e, d):
    assert m % NWIN == 0
    win = m // NWIN
    cpr = 40
    assert win % cpr == 0 and cpr % 8 == 0
    nch = win // cpr
    kmax = (nch + NS - 1) // NS
    assert e % NS == 0
    eps = e // NS
    assert eps % BLK == 0
    nblk = eps // BLK

    mesh = plsc.VectorSubcoreMesh(core_axis_name="c", subcore_axis_name="s")

    @functools.partial(
        pl.kernel,
        out_type=jax.ShapeDtypeStruct((m, d), jnp.float32),
        mesh=mesh,
        scratch_types=[
            pltpu.VMEM((2, BLK), jnp.int32),
            pltpu.VMEM((2, BLK), jnp.int32),
            pltpu.VMEM((CAP,), jnp.int32),
            pltpu.VMEM((CAP,), jnp.int32),
            pltpu.VMEM((NB, CHUNK), jnp.int32),
            pltpu.VMEM((NB, CHUNK, d), jnp.float32),
            pltpu.VMEM((cpr, d), jnp.float32),
            pltpu.VMEM_SHARED((win + 1, d), jnp.float32),
            pltpu.SemaphoreType.DMA,
            pltpu.SemaphoreType.DMA,
            pltpu.SemaphoreType.DMA,
        ],
        compiler_params=pltpu.CompilerParams(needs_layout_passes=False),
    )
    def msg_pass(tx_hbm, src_hbm, dst_hbm, out_hbm,
                 src_v, dst_v, gidx, lidx, schunk, rows_v, zero_v,
                 acc, esem, gsem, ssem):
        c = lax.axis_index("c")
        s = lax.axis_index("s")
        zvec = jnp.zeros((L,), jnp.float32)

        def zrow(i, carry):
            for j in range(d // L):
                zero_v[i, pl.ds(j * L, L)] = zvec
            return carry

        lax.fori_loop(0, cpr, zrow, 0)

        edge_base = s * eps

        def start_block(bi, buf):
            off = edge_base + bi * BLK
            pltpu.async_copy(src_hbm.at[pl.ds(off, BLK)], src_v.at[buf],
                             esem)
            pltpu.async_copy(dst_hbm.at[pl.ds(off, BLK)], dst_v.at[buf],
                             esem)

        def wait_block(buf):
            pltpu.make_async_copy(src_hbm.at[pl.ds(0, BLK)], src_v.at[buf],
                                  esem).wait()
            pltpu.make_async_copy(dst_hbm.at[pl.ds(0, BLK)], dst_v.at[buf],
                                  esem).wait()

        def stage_sidx(q, o):
            for j in range(CHUNK // L):
                schunk[q, pl.ds(j * L, L)] = lidx[pl.ds(o + j * L, L)]

        def burst_body(t, carry):
            o = t * BURST
            for q in range(NB):
                stage_sidx(q, o + q * CHUNK)
                pltpu.async_copy(
                    tx_hbm.at[gidx.at[pl.ds(o + q * CHUNK, CHUNK)]],
                    rows_v.at[q], gsem)
            sdescs = []
            for q in range(NB):
                pltpu.make_async_copy(
                    tx_hbm.at[gidx.at[pl.ds(o + q * CHUNK, CHUNK)]],
                    rows_v.at[q], gsem).wait()
                sdescs.append(
                    pltpu.async_copy(rows_v.at[q], acc.at[schunk.at[q]],
                                     ssem, add=True))
            for sd in sdescs:
                sd.wait()
            return carry

        def drain_one(i, carry):
            o = i * CHUNK
            stage_sidx(0, o)
            pltpu.async_copy(tx_hbm.at[gidx.at[pl.ds(o, CHUNK)]],
                             rows_v.at[0], gsem).wait()
            pltpu.sync_copy(rows_v.at[0], acc.at[schunk.at[0]], add=True)
            return carry

        def window_body(wi, carry):
            w = wi * NC + c
            base = w * win
            basev = jnp.full((L,), base, jnp.int32)
            winu = jnp.full((L,), win, jnp.uint32)

            for k in range(kmax):
                idx = s + k * NS

                @pl.when(idx < nch)
                def _zero_chunk():
                    pltpu.async_copy(zero_v, acc.at[pl.ds(idx * cpr, cpr)],
                                     esem)
            for k in range(kmax):
                idx = s + k * NS

                @pl.when(idx < nch)
                def _zero_wait():
                    pltpu.make_async_copy(
                        zero_v, acc.at[pl.ds(idx * cpr, cpr)], esem).wait()
            plsc.subcore_barrier()

            start_block(0, 0)

            def block_body(bi, cnt):
                par = lax.rem(bi, 2)
                wait_block(par)

                @pl.when(bi + 1 < nblk)
                def _prefetch():
                    start_block(bi + 1, 1 - par)

                def vec_body(j, cnt):
                    U = 8
                    dv = [dst_v[par, pl.ds(j * U * L + u * L, L)]
                          for u in range(U)]
                    sv = [src_v[par, pl.ds(j * U * L + u * L, L)]
                          for u in range(U)]
                    lo = [dv[u] - basev for u in range(U)]
                    mk = [lo[u].astype(jnp.uint32) < winu for u in range(U)]
                    cs = [plsc.cumsum(mk[u].astype(jnp.int32))
                          for u in range(U)]
                    off = cnt - 1
                    for u in range(U):
                        pos = cs[u] + jnp.full((L,), off, jnp.int32)
                        plsc.store_scatter(gidx, [pos], sv[u], mask=mk[u])
                        plsc.store_scatter(lidx, [pos], lo[u], mask=mk[u])
                        off = off + cs[u][L - 1]
                    return off + 1

                cnt = lax.fori_loop(0, BLK // (8 * L), vec_body, cnt)

                nburst = cnt // BURST
                lax.fori_loop(0, nburst, burst_body, 0)

                @pl.when(nburst > 0)
                def _move_leftover():
                    rem = nburst * BURST
                    for j in range(BURST // L):
                        gv = gidx[pl.ds(rem + j * L, L)]
                        lv = lidx[pl.ds(rem + j * L, L)]
                        gidx[pl.ds(j * L, L)] = gv
                        lidx[pl.ds(j * L, L)] = lv

                return cnt - nburst * BURST

            cnt = lax.fori_loop(0, nblk, block_body, 0)

            @pl.when(cnt > 0)
            def _final_drain():
                for j in range(CHUNK // L):
                    gidx[pl.ds(cnt + j * L, L)] = jnp.zeros((L,), jnp.int32)
                    lidx[pl.ds(cnt + j * L, L)] = jnp.full((L,), win,
                                                           jnp.int32)
                nfin = (cnt + CHUNK - 1) // CHUNK
                lax.fori_loop(0, nfin, drain_one, 0)

            plsc.subcore_barrier()

            for k in range(kmax):
                idx = s + k * NS

                @pl.when(idx < nch)
                def _copy_chunk():
                    r = idx * cpr
                    pltpu.async_copy(
                        acc.at[pl.ds(r, cpr)],
                        out_hbm.at[pl.ds(base + r, cpr)], esem)
            for k in range(kmax):
                idx = s + k * NS

                @pl.when(idx < nch)
                def _copy_wait():
                    r = idx * cpr
                    pltpu.make_async_copy(
                        acc.at[pl.ds(r, cpr)],
                        out_hbm.at[pl.ds(base + r, cpr)], esem).wait()
            return carry

        lax.fori_loop(0, NWIN // NC, window_body, 0)

    return msg_pass


def kernel(x_vals, msg_src, msg_dst, W, b):
    m, d = x_vals.shape[0], W.shape[1]
    tx = _matmul(x_vals, W, b)
    msg_pass = _make_msg_pass(m, msg_src.shape[0], d)
    return msg_pass(tx, msg_src, msg_dst)
